# baseline (speedup 1.0000x reference)
import functools

import jax
import jax.numpy as jnp
from jax import lax
from jax.experimental import pallas as pl
from jax.experimental.pallas import tpu as pltpu
from jax.experimental.pallas import tpu_sc as plsc

EMBED = 64
SCALE = 8.0
LANES = 16
CHUNK = 320
NWORK = 32


def kernel(x, table):
    B, L = x.shape
    N = B * L
    per_w = N // NWORK
    n_chunks = per_w // CHUNK
    rows_w = per_w // CHUNK

    idx = x.reshape(N // CHUNK, CHUNK)

    mesh = plsc.VectorSubcoreMesh(core_axis_name="c", subcore_axis_name="s")

    @functools.partial(
        pl.kernel,
        out_type=jax.ShapeDtypeStruct((N, 2 * EMBED), jnp.float32),
        mesh=mesh,
        compiler_params=pltpu.CompilerParams(use_tc_tiling_on_sc=False),
        scratch_types=[
            pltpu.VMEM((rows_w, CHUNK), jnp.int32),
            pltpu.VMEM((2, CHUNK, EMBED), jnp.float32),
            pltpu.VMEM((2, CHUNK, EMBED), jnp.float32),
            pltpu.SemaphoreType.DMA,
            pltpu.SemaphoreType.DMA,
            pltpu.SemaphoreType.DMA,
            pltpu.SemaphoreType.DMA,
            pltpu.SemaphoreType.DMA,
        ],
    )
    def gather_scale(i_hbm, t_hbm, o_hbm, iv, gb, ob, sem_i, sg0, sg1, so0,
                     so1):
        wid = lax.axis_index("s") * 2 + lax.axis_index("c")
        base = wid * per_w
        sem_g = (sg0, sg1)
        sem_o = (so0, so1)

        pltpu.async_copy(
            i_hbm.at[pl.ds(wid * rows_w, rows_w), :], iv, sem_i
        ).wait()

        def start_gather(chunk, slot):
            pltpu.async_copy(t_hbm.at[iv.at[chunk]], gb.at[slot], sem_g[slot])

        def wait_gather(slot):
            pltpu.make_async_copy(
                t_hbm.at[iv.at[0]], gb.at[slot], sem_g[slot]).wait()

        def scale(slot):
            @pl.loop(0, CHUNK)
            def _r(r):
                for k in range(EMBED // LANES):
                    sl = pl.ds(k * LANES, LANES)
                    ob.at[slot, r, sl][...] = gb.at[slot, r, sl][...] * SCALE

        def start_out(chunk, slot):
            pltpu.async_copy(
                ob.at[slot],
                o_hbm.at[pl.ds(base + chunk * CHUNK, CHUNK), pl.ds(0, EMBED)],
                sem_o[slot],
            )

        def wait_out(chunk, slot):
            pltpu.make_async_copy(
                ob.at[slot],
                o_hbm.at[pl.ds(base + chunk * CHUNK, CHUNK), pl.ds(0, EMBED)],
                sem_o[slot],
            ).wait()

        start_gather(0, 0)

        @pl.loop(0, n_chunks // 2)
        def _t(t):
            for b in range(2):
                i = t * 2 + b
                nxt = 1 - b

                @pl.when(i + 1 < n_chunks)
                def _():
                    start_gather(i + 1, nxt)

                wait_gather(b)

                @pl.when(i >= 2)
                def _():
                    wait_out(i - 2, b)

                scale(b)
                start_out(i, b)

        for b in range(2):
            wait_out(n_chunks - 2 + b, b)

    out = gather_scale(idx, table)
    return out[:, :EMBED].reshape(B, L, EMBED)
